# baseline (speedup 1.0000x reference)
import functools

import jax
import jax.numpy as jnp
from jax import lax
from jax.experimental import pallas as pl
from jax.experimental.pallas import tpu as pltpu
from jax.experimental.pallas import tpu_sc as plsc

N = 10000
E = 160000
HID = 128
H = 8
C = 16
NG = 64

_NC = 2
_NS = 16
_NW = _NC * _NS
_CE = 128

_BN = 2000
_BE = 2000


def _mm(a, w):
    return lax.dot_general(a, w, (((1,), (1,)), ((), ())),
                           preferred_element_type=jnp.float32)


def _rowlin_body(x_ref, w_ref, b_ref, o_ref, *, relu):
    y = _mm(x_ref[...], w_ref[...]) + b_ref[...]
    if relu:
        y = jnp.maximum(y, 0.0)
    o_ref[...] = y


def _rowlin(x, wb, relu, block):
    w, b = wb
    rows, din = x.shape
    dout = w.shape[0]
    grid = rows // block
    return pl.pallas_call(
        functools.partial(_rowlin_body, relu=relu),
        grid=(grid,),
        in_specs=[
            pl.BlockSpec((block, din), lambda i: (i, 0)),
            pl.BlockSpec((dout, din), lambda i: (0, 0)),
            pl.BlockSpec((1, dout), lambda i: (0, 0)),
        ],
        out_specs=pl.BlockSpec((block, dout), lambda i: (i, 0)),
        out_shape=jax.ShapeDtypeStruct((rows, dout), jnp.float32),
    )(x, w, b.reshape(1, dout))


def _edge_body(ea_ref, w1_ref, b1_ref, w2_ref, b2_ref, we_ref, ean_ref, e_ref):
    t = jnp.maximum(_mm(ea_ref[...], w1_ref[...]) + b1_ref[...], 0.0)
    ean = _mm(t, w2_ref[...]) + b2_ref[...]
    ean_ref[...] = ean
    e_ref[...] = _mm(ean, we_ref[...])


def _edge_layer(ea, p):
    w1, b1 = p['em1']
    w2, b2 = p['em2']
    we = p['e']
    grid = E // _BE
    full = lambda i: (0, 0)
    return pl.pallas_call(
        _edge_body,
        grid=(grid,),
        in_specs=[
            pl.BlockSpec((_BE, HID), lambda i: (i, 0)),
            pl.BlockSpec((HID, HID), full),
            pl.BlockSpec((1, HID), full),
            pl.BlockSpec((HID, HID), full),
            pl.BlockSpec((1, HID), full),
            pl.BlockSpec((HID, HID), full),
        ],
        out_specs=[
            pl.BlockSpec((_BE, HID), lambda i: (i, 0)),
            pl.BlockSpec((_BE, HID), lambda i: (i, 0)),
        ],
        out_shape=[
            jax.ShapeDtypeStruct((E, HID), jnp.float32),
            jax.ShapeDtypeStruct((E, HID), jnp.float32),
        ],
    )(ea, w1, b1.reshape(1, HID), w2, b2.reshape(1, HID), we)


def _qkv_body(h_ref, wq_ref, bq_ref, wk_ref, bk_ref, wv_ref, bv_ref,
              q_ref, k_ref, v_ref):
    h = h_ref[...]
    q_ref[...] = _mm(h, wq_ref[...]) + bq_ref[...]
    k_ref[...] = _mm(h, wk_ref[...]) + bk_ref[...]
    v_ref[...] = _mm(h, wv_ref[...]) + bv_ref[...]


def _qkv(h, p):
    wq, bq = p['q']
    wk, bk = p['k']
    wv, bv = p['v']
    grid = N // _BN
    full = lambda i: (0, 0)
    blk = pl.BlockSpec((_BN, HID), lambda i: (i, 0))
    return pl.pallas_call(
        _qkv_body,
        grid=(grid,),
        in_specs=[blk,
                  pl.BlockSpec((HID, HID), full), pl.BlockSpec((1, HID), full),
                  pl.BlockSpec((HID, HID), full), pl.BlockSpec((1, HID), full),
                  pl.BlockSpec((HID, HID), full), pl.BlockSpec((1, HID), full)],
        out_specs=[blk, blk, blk],
        out_shape=[jax.ShapeDtypeStruct((N, HID), jnp.float32)] * 3,
    )(h, wq, bq.reshape(1, HID), wk, bk.reshape(1, HID), wv, bv.reshape(1, HID))


_NCHUNKS = E // _CE

_GDN = lax.GatherDimensionNumbers(offset_dims=(), collapsed_slice_dims=(0,),
                                  start_index_map=(0,))


def _lane_shuffle(x, perm):
    return lax.gather(x, perm[:, None], _GDN, (1,),
                      mode=lax.GatherScatterMode.PROMISE_IN_BOUNDS)


def _lane_sum(x, perms):
    for perm in perms:
        x = x + _lane_shuffle(x, perm)
    return x


_NP = 10240
_ROWS_PER_S = _NP // _NS


_HH = H // _NC
_HW = _HH * C


def _sc_attn_body(q_hbm, k_hbm, v_hbm, e_hbm, src_hbm, dst_hbm,
                  zm_hbm, zw_hbm, outm_hbm, outw_hbm,
                  srcb, dstb, idxb, idx2b, qb, kb, vb, eb, wb,
                  accm, accw, sem):
    cid = lax.axis_index("c")
    sid = lax.axis_index("s")

    r0 = pl.multiple_of(sid * _ROWS_PER_S, 8)
    pltpu.sync_copy(zm_hbm.at[pl.ds(r0, _ROWS_PER_S)],
                    accm.at[pl.ds(r0, _ROWS_PER_S)])
    pltpu.sync_copy(zw_hbm.at[pl.ds(r0, _ROWS_PER_S)],
                    accw.at[pl.ds(r0, _ROWS_PER_S)])
    plsc.subcore_barrier()

    base = _NCHUNKS // _NS
    extra = _NCHUNKS - base * _NS
    nch = base + jnp.where(sid < extra, 1, 0)
    lane = lax.iota(jnp.int32, 16)
    perms = [lane ^ m for m in (8, 4, 2, 1)]

    def chunk_body(t, carry):
        off = pl.multiple_of((sid + t * _NS) * _CE, _CE)
        pltpu.sync_copy(src_hbm.at[pl.ds(off, _CE)], srcb)
        pltpu.sync_copy(dst_hbm.at[pl.ds(off, _CE)], dstb)

        @plsc.parallel_loop(0, _CE // 16, unroll=4)
        def _(j):
            sl = pl.ds(j * 16, 16)
            idxb[sl] = srcb[sl] * 2 + cid
            idx2b[sl] = dstb[sl] * 2 + cid

        cp_k = pltpu.async_copy(k_hbm.at[idxb], kb, sem)
        cp_v = pltpu.async_copy(v_hbm.at[idxb], vb, sem)
        cp_q = pltpu.async_copy(q_hbm.at[idx2b], qb, sem)
        pltpu.sync_copy(e_hbm.at[pl.ds(off, _CE), cid], eb)
        cp_k.wait()
        cp_v.wait()
        cp_q.wait()

        @plsc.parallel_loop(0, _CE, unroll=8)
        def _(i):
            wacc = jnp.zeros((C,), jnp.float32)
            for hh in range(_HH):
                sl = pl.ds(hh * C, C)
                qh = qb[i, sl]
                kj = kb[i, sl] + eb[i, sl]
                a = _lane_sum(qh * kj, perms)
                wv = jnp.exp(a * 0.25)
                vb[i, sl] = (vb[i, sl] + eb[i, sl]) * wv
                wacc = jnp.where(lane == hh, wv, wacc)
            wb[i, :] = wacc
        pltpu.sync_copy(vb, accm.at[dstb], add=True)
        pltpu.sync_copy(wb, accw.at[dstb], add=True)
        return carry

    lax.fori_loop(0, nch, chunk_body, 0)
    plsc.subcore_barrier()

    pltpu.sync_copy(accm.at[pl.ds(r0, _ROWS_PER_S)],
                    outm_hbm.at[cid, pl.ds(r0, _ROWS_PER_S)])
    pltpu.sync_copy(accw.at[pl.ds(r0, _ROWS_PER_S)],
                    outw_hbm.at[cid, pl.ds(r0, _ROWS_PER_S)])


def _sc_attn(q2, k2, v2, e3, src, dst, zm, zw):
    mesh = plsc.VectorSubcoreMesh(core_axis_name="c", subcore_axis_name="s")
    fn = functools.partial(
        pl.kernel, mesh=mesh,
        compiler_params=pltpu.CompilerParams(use_tc_tiling_on_sc=False),
        out_type=[jax.ShapeDtypeStruct((_NC, _NP, _HW), jnp.float32),
                  jax.ShapeDtypeStruct((_NC, _NP, C), jnp.float32)],
        scratch_types=[
            pltpu.VMEM((_CE,), jnp.int32),
            pltpu.VMEM((_CE,), jnp.int32),
            pltpu.VMEM((_CE,), jnp.int32),
            pltpu.VMEM((_CE,), jnp.int32),
            pltpu.VMEM((_CE, _HW), jnp.float32),
            pltpu.VMEM((_CE, _HW), jnp.float32),
            pltpu.VMEM((_CE, _HW), jnp.float32),
            pltpu.VMEM((_CE, _HW), jnp.float32),
            pltpu.VMEM((_CE, C), jnp.float32),
            pltpu.VMEM_SHARED((_NP, _HW), jnp.float32),
            pltpu.VMEM_SHARED((_NP, C), jnp.float32),
            pltpu.SemaphoreType.DMA,
        ],
    )(_sc_attn_body)
    return fn(q2, k2, v2, e3, src, dst, zm, zw)


def _post_body(mp_ref, wp_ref, h_ref, wsk_ref, bsk_ref, beta_ref,
               w1_ref, b1_ref, w2_ref, b2_ref, g_ref, bln_ref, o_ref):
    msg = jnp.concatenate([mp_ref[0], mp_ref[1]], axis=-1)
    wcat = jnp.concatenate([wp_ref[0], wp_ref[1]], axis=-1)
    rows = lax.broadcasted_iota(jnp.int32, (2 * C, HID), 0)
    gh = lax.broadcasted_iota(jnp.int32, (2 * C, HID), 1) // C
    expand = (((rows == gh) & (gh < _HH)) |
              ((rows == gh + 12) & (gh >= _HH))).astype(jnp.float32)
    den = lax.dot_general(wcat, expand, (((1,), (0,)), ((), ())),
                          preferred_element_type=jnp.float32)
    out = msg / (den + 1e-16)

    h = h_ref[...]
    xr = _mm(h, wsk_ref[...]) + bsk_ref[...]
    bvec = beta_ref[...]
    ba = bvec[:, 0:HID]
    bb = bvec[:, HID:2 * HID]
    bc = bvec[:, 2 * HID:3 * HID]
    logits = (_mm(out, ba) + _mm(xr, bb) + _mm(out - xr, bc))
    bt = jax.nn.sigmoid(logits)
    out = bt * xr + (1.0 - bt) * out

    t = jnp.maximum(_mm(out, w1_ref[...]) + b1_ref[...], 0.0)
    h2 = _mm(t, w2_ref[...]) + b2_ref[...] + h
    m = jnp.mean(h2, axis=-1, keepdims=True)
    d = h2 - m
    var = jnp.mean(d * d, axis=-1, keepdims=True)
    o_ref[...] = d * lax.rsqrt(var + 1e-5) * g_ref[...] + bln_ref[...]


def _post(mp, wp, h, p):
    wsk, bsk = p['skip']
    w1, b1 = p['nm1']
    w2, b2 = p['nm2']
    g, bln = p['ln']
    grid = N // _BN
    full = lambda i: (0, 0)
    blk = pl.BlockSpec((_BN, HID), lambda i: (i, 0))
    return pl.pallas_call(
        _post_body,
        grid=(grid,),
        in_specs=[
            pl.BlockSpec((_NC, _BN, _HW), lambda i: (0, i, 0)),
            pl.BlockSpec((_NC, _BN, C), lambda i: (0, i, 0)),
            blk,
            pl.BlockSpec((HID, HID), full), pl.BlockSpec((1, HID), full),
            pl.BlockSpec((1, 3 * HID), full),
            pl.BlockSpec((HID, HID), full), pl.BlockSpec((1, HID), full),
            pl.BlockSpec((HID, HID), full), pl.BlockSpec((1, HID), full),
            pl.BlockSpec((1, HID), full), pl.BlockSpec((1, HID), full),
        ],
        out_specs=blk,
        out_shape=jax.ShapeDtypeStruct((N, HID), jnp.float32),
    )(mp, wp, h, wsk, bsk.reshape(1, HID), p['beta'],
      w1, b1.reshape(1, HID), w2, b2.reshape(1, HID),
      g.reshape(1, HID), bln.reshape(1, HID))


def _pool_body(b_ref, h_ref, wp1_ref, bp1_ref, wp2_ref, bp2_ref, o_ref,
               acc, cacc, *, grid):
    i = pl.program_id(0)

    @pl.when(i == 0)
    def _():
        acc[...] = jnp.zeros_like(acc)
        cacc[...] = jnp.zeros_like(cacc)

    bids = b_ref[0, 0, :]
    gid = lax.broadcasted_iota(jnp.int32, (NG, bids.shape[0]), 0)
    onehot = (gid == bids[None, :]).astype(jnp.float32)
    acc[...] += lax.dot_general(onehot, h_ref[...], (((1,), (0,)), ((), ())),
                                preferred_element_type=jnp.float32)
    cnt = jnp.sum(onehot, axis=1, keepdims=True)
    cacc[...] += jnp.broadcast_to(cnt, (NG, HID))

    @pl.when(i == grid - 1)
    def _():
        gmean = acc[...] / jnp.maximum(cacc[...], 1.0)
        t = jnp.maximum(_mm(gmean, wp1_ref[...]) + bp1_ref[...], 0.0)
        o_ref[...] = _mm(t, wp2_ref[...]) + bp2_ref[0, 0]


def _pool_head(h, batch3, params):
    wp1, bp1 = params['pred1']
    wp2, bp2 = params['pred2']
    hid2 = wp1.shape[0]
    nt = wp2.shape[0]
    wp2p = jnp.pad(wp2, ((0, HID - nt), (0, 0)))
    bn = batch3.shape[2]
    grid = N // bn
    full = lambda i: (0, 0)
    res = pl.pallas_call(
        functools.partial(_pool_body, grid=grid),
        grid=(grid,),
        in_specs=[
            pl.BlockSpec((1, 1, bn), lambda i: (i, 0, 0)),
            pl.BlockSpec((bn, HID), lambda i: (i, 0)),
            pl.BlockSpec((hid2, HID), full),
            pl.BlockSpec((1, hid2), full),
            pl.BlockSpec((HID, hid2), full),
            pl.BlockSpec((1, nt), full),
        ],
        out_specs=pl.BlockSpec((NG, HID), lambda i: (0, 0)),
        out_shape=jax.ShapeDtypeStruct((NG, HID), jnp.float32),
        scratch_shapes=[pltpu.VMEM((NG, HID), jnp.float32),
                        pltpu.VMEM((NG, HID), jnp.float32)],
    )(batch3, h, wp1, bp1.reshape(1, hid2), wp2p, bp2.reshape(1, nt))
    return res[:, :nt]


def kernel(x, edge_index, edge_attr, batch, params):
    src = edge_index[0]
    dst = edge_index[1]
    h = _rowlin(x, params['node_enc'], relu=True, block=_BN)
    ea = _rowlin(edge_attr, params['edge_enc'], relu=False, block=_BE)
    zm = jnp.zeros((_NP, _HW), jnp.float32)
    zw = jnp.zeros((_NP, C), jnp.float32)
    batch3 = batch.reshape(10, 1, N // 10)
    for p in params['layers']:
        ea, e = _edge_layer(ea, p)
        q, k, v = _qkv(h, p)
        q2 = q.reshape(N, _NC, _HW).reshape(_NC * N, _HW)
        k2 = k.reshape(N, _NC, _HW).reshape(_NC * N, _HW)
        v2 = v.reshape(N, _NC, _HW).reshape(_NC * N, _HW)
        e3 = e.reshape(E, _NC, _HW)
        mp, wp = _sc_attn(q2, k2, v2, e3, src, dst, zm, zw)
        h = _post(mp, wp, h, p)
    return _pool_head(h, batch3, params)


# speedup vs baseline: 1.3434x; 1.3434x over previous
import functools

import jax
import jax.numpy as jnp
from jax import lax
from jax.experimental import pallas as pl
from jax.experimental.pallas import tpu as pltpu
from jax.experimental.pallas import tpu_sc as plsc

N = 10000
E = 160000
HID = 128
H = 8
C = 16
NG = 64

_NC = 2
_NS = 16
_NW = _NC * _NS
_CE = 128

_BN = 2000
_BE = 2000
_HW0 = HID // _NC


def _mm(a, w):
    return lax.dot_general(a, w, (((1,), (1,)), ((), ())),
                           preferred_element_type=jnp.float32)


def _rowlin_body(x_ref, w_ref, b_ref, o_ref, *, relu):
    y = _mm(x_ref[...], w_ref[...]) + b_ref[...]
    if relu:
        y = jnp.maximum(y, 0.0)
    o_ref[...] = y


def _rowlin(x, wb, relu, block):
    w, b = wb
    rows, din = x.shape
    dout = w.shape[0]
    grid = rows // block
    return pl.pallas_call(
        functools.partial(_rowlin_body, relu=relu),
        grid=(grid,),
        in_specs=[
            pl.BlockSpec((block, din), lambda i: (i, 0)),
            pl.BlockSpec((dout, din), lambda i: (0, 0)),
            pl.BlockSpec((1, dout), lambda i: (0, 0)),
        ],
        out_specs=pl.BlockSpec((block, dout), lambda i: (i, 0)),
        out_shape=jax.ShapeDtypeStruct((rows, dout), jnp.float32),
    )(x, w, b.reshape(1, dout))


def _edge_body(ea_ref, w1_ref, b1_ref, w2_ref, b2_ref, we_ref, ean_ref, e_ref):
    t = jnp.maximum(_mm(ea_ref[...], w1_ref[...]) + b1_ref[...], 0.0)
    ean = _mm(t, w2_ref[...]) + b2_ref[...]
    ean_ref[...] = ean
    e = _mm(ean, we_ref[...])
    e_ref[0] = e[:, :_HW0]
    e_ref[1] = e[:, _HW0:]


def _edge_layer(ea, p):
    w1, b1 = p['em1']
    w2, b2 = p['em2']
    we = p['e']
    grid = E // _BE
    full = lambda i: (0, 0)
    return pl.pallas_call(
        _edge_body,
        grid=(grid,),
        in_specs=[
            pl.BlockSpec((_BE, HID), lambda i: (i, 0)),
            pl.BlockSpec((HID, HID), full),
            pl.BlockSpec((1, HID), full),
            pl.BlockSpec((HID, HID), full),
            pl.BlockSpec((1, HID), full),
            pl.BlockSpec((HID, HID), full),
        ],
        out_specs=[
            pl.BlockSpec((_BE, HID), lambda i: (i, 0)),
            pl.BlockSpec((_NC, _BE, _HW0), lambda i: (0, i, 0)),
        ],
        out_shape=[
            jax.ShapeDtypeStruct((E, HID), jnp.float32),
            jax.ShapeDtypeStruct((_NC, E, _HW0), jnp.float32),
        ],
    )(ea, w1, b1.reshape(1, HID), w2, b2.reshape(1, HID), we)


def _qkv_body(h_ref, wq_ref, bq_ref, wk_ref, bk_ref, wv_ref, bv_ref,
              q_ref, k_ref, v_ref):
    h = h_ref[...]
    for w_ref, b_ref, o_ref in ((wq_ref, bq_ref, q_ref),
                                (wk_ref, bk_ref, k_ref),
                                (wv_ref, bv_ref, v_ref)):
        y = _mm(h, w_ref[...]) + b_ref[...]
        o_ref[0] = y[:, :_HW0]
        o_ref[1] = y[:, _HW0:]


def _qkv(h, p):
    wq, bq = p['q']
    wk, bk = p['k']
    wv, bv = p['v']
    grid = N // _BN
    full = lambda i: (0, 0)
    blk = pl.BlockSpec((_BN, HID), lambda i: (i, 0))
    oblk = pl.BlockSpec((_NC, _BN, _HW0), lambda i: (0, i, 0))
    return pl.pallas_call(
        _qkv_body,
        grid=(grid,),
        in_specs=[blk,
                  pl.BlockSpec((HID, HID), full), pl.BlockSpec((1, HID), full),
                  pl.BlockSpec((HID, HID), full), pl.BlockSpec((1, HID), full),
                  pl.BlockSpec((HID, HID), full), pl.BlockSpec((1, HID), full)],
        out_specs=[oblk, oblk, oblk],
        out_shape=[jax.ShapeDtypeStruct((_NC, N, _HW0), jnp.float32)] * 3,
    )(h, wq, bq.reshape(1, HID), wk, bk.reshape(1, HID), wv, bv.reshape(1, HID))


_NCHUNKS = E // _CE

_GDN = lax.GatherDimensionNumbers(offset_dims=(), collapsed_slice_dims=(0,),
                                  start_index_map=(0,))


def _lane_shuffle(x, perm):
    return lax.gather(x, perm[:, None], _GDN, (1,),
                      mode=lax.GatherScatterMode.PROMISE_IN_BOUNDS)


def _lane_sum(x, perms):
    for perm in perms:
        x = x + _lane_shuffle(x, perm)
    return x


_NP = 10240
_ROWS_PER_S = _NP // _NS


_HH = H // _NC
_HW = _HH * C


def _sc_attn_body(q_hbm, k_hbm, v_hbm, e_hbm, src_hbm, dst_hbm,
                  zm_hbm, zw_hbm, outm_hbm, outw_hbm,
                  srcb, dstb, idxb, idx2b, qb, kb, vb, eb, wb,
                  accm, accw, sem):
    cid = lax.axis_index("c")
    sid = lax.axis_index("s")

    r0 = pl.multiple_of(sid * _ROWS_PER_S, 8)
    pltpu.sync_copy(zm_hbm.at[pl.ds(r0, _ROWS_PER_S)],
                    accm.at[pl.ds(r0, _ROWS_PER_S)])
    pltpu.sync_copy(zw_hbm.at[pl.ds(r0, _ROWS_PER_S)],
                    accw.at[pl.ds(r0, _ROWS_PER_S)])
    plsc.subcore_barrier()

    base = _NCHUNKS // _NS
    extra = _NCHUNKS - base * _NS
    nch = base + jnp.where(sid < extra, 1, 0)
    lane = lax.iota(jnp.int32, 16)
    perms = [lane ^ m for m in (8, 4, 2, 1)]

    def chunk_body(t, carry):
        off = pl.multiple_of((sid + t * _NS) * _CE, _CE)
        pltpu.sync_copy(src_hbm.at[pl.ds(off, _CE)], srcb)
        pltpu.sync_copy(dst_hbm.at[pl.ds(off, _CE)], dstb)

        @plsc.parallel_loop(0, _CE // 16, unroll=4)
        def _(j):
            sl = pl.ds(j * 16, 16)
            idxb[sl] = srcb[sl] + cid * N
            idx2b[sl] = dstb[sl] + cid * N

        cp_k = pltpu.async_copy(k_hbm.at[idxb], kb, sem)
        cp_v = pltpu.async_copy(v_hbm.at[idxb], vb, sem)
        cp_q = pltpu.async_copy(q_hbm.at[idx2b], qb, sem)
        pltpu.sync_copy(e_hbm.at[cid, pl.ds(off, _CE)], eb)
        cp_k.wait()
        cp_v.wait()
        cp_q.wait()

        @plsc.parallel_loop(0, _CE, unroll=4)
        def _(i):
            wacc = jnp.zeros((C,), jnp.float32)
            for hh in range(_HH):
                sl = pl.ds(hh * C, C)
                qh = qb[i, sl]
                kj = kb[i, sl] + eb[i, sl]
                a = _lane_sum(qh * kj, perms)
                wv = jnp.exp(a * 0.25)
                vb[i, sl] = (vb[i, sl] + eb[i, sl]) * wv
                wacc = jnp.where(lane == hh, wv, wacc)
            wb[i, :] = wacc
        pltpu.sync_copy(vb, accm.at[dstb], add=True)
        pltpu.sync_copy(wb, accw.at[dstb], add=True)
        return carry

    lax.fori_loop(0, nch, chunk_body, 0)
    plsc.subcore_barrier()

    pltpu.sync_copy(accm.at[pl.ds(r0, _ROWS_PER_S)],
                    outm_hbm.at[cid, pl.ds(r0, _ROWS_PER_S)])
    pltpu.sync_copy(accw.at[pl.ds(r0, _ROWS_PER_S)],
                    outw_hbm.at[cid, pl.ds(r0, _ROWS_PER_S)])


def _sc_attn(q2, k2, v2, e3, src, dst, zm, zw):
    mesh = plsc.VectorSubcoreMesh(core_axis_name="c", subcore_axis_name="s")
    fn = functools.partial(
        pl.kernel, mesh=mesh,
        compiler_params=pltpu.CompilerParams(use_tc_tiling_on_sc=False),
        out_type=[jax.ShapeDtypeStruct((_NC, _NP, _HW), jnp.float32),
                  jax.ShapeDtypeStruct((_NC, _NP, C), jnp.float32)],
        scratch_types=[
            pltpu.VMEM((_CE,), jnp.int32),
            pltpu.VMEM((_CE,), jnp.int32),
            pltpu.VMEM((_CE,), jnp.int32),
            pltpu.VMEM((_CE,), jnp.int32),
            pltpu.VMEM((_CE, _HW), jnp.float32),
            pltpu.VMEM((_CE, _HW), jnp.float32),
            pltpu.VMEM((_CE, _HW), jnp.float32),
            pltpu.VMEM((_CE, _HW), jnp.float32),
            pltpu.VMEM((_CE, C), jnp.float32),
            pltpu.VMEM_SHARED((_NP, _HW), jnp.float32),
            pltpu.VMEM_SHARED((_NP, C), jnp.float32),
            pltpu.SemaphoreType.DMA,
        ],
    )(_sc_attn_body)
    return fn(q2, k2, v2, e3, src, dst, zm, zw)


def _post_body(mp_ref, wp_ref, h_ref, wsk_ref, bsk_ref, beta_ref,
               w1_ref, b1_ref, w2_ref, b2_ref, g_ref, bln_ref, o_ref):
    msg = jnp.concatenate([mp_ref[0], mp_ref[1]], axis=-1)
    wcat = jnp.concatenate([wp_ref[0], wp_ref[1]], axis=-1)
    rows = lax.broadcasted_iota(jnp.int32, (2 * C, HID), 0)
    gh = lax.broadcasted_iota(jnp.int32, (2 * C, HID), 1) // C
    expand = (((rows == gh) & (gh < _HH)) |
              ((rows == gh + 12) & (gh >= _HH))).astype(jnp.float32)
    den = lax.dot_general(wcat, expand, (((1,), (0,)), ((), ())),
                          preferred_element_type=jnp.float32)
    out = msg / (den + 1e-16)

    h = h_ref[...]
    xr = _mm(h, wsk_ref[...]) + bsk_ref[...]
    bvec = beta_ref[...]
    ba = bvec[:, 0:HID]
    bb = bvec[:, HID:2 * HID]
    bc = bvec[:, 2 * HID:3 * HID]
    logits = (_mm(out, ba) + _mm(xr, bb) + _mm(out - xr, bc))
    bt = jax.nn.sigmoid(logits)
    out = bt * xr + (1.0 - bt) * out

    t = jnp.maximum(_mm(out, w1_ref[...]) + b1_ref[...], 0.0)
    h2 = _mm(t, w2_ref[...]) + b2_ref[...] + h
    m = jnp.mean(h2, axis=-1, keepdims=True)
    d = h2 - m
    var = jnp.mean(d * d, axis=-1, keepdims=True)
    o_ref[...] = d * lax.rsqrt(var + 1e-5) * g_ref[...] + bln_ref[...]


def _post(mp, wp, h, p):
    wsk, bsk = p['skip']
    w1, b1 = p['nm1']
    w2, b2 = p['nm2']
    g, bln = p['ln']
    grid = N // _BN
    full = lambda i: (0, 0)
    blk = pl.BlockSpec((_BN, HID), lambda i: (i, 0))
    return pl.pallas_call(
        _post_body,
        grid=(grid,),
        in_specs=[
            pl.BlockSpec((_NC, _BN, _HW), lambda i: (0, i, 0)),
            pl.BlockSpec((_NC, _BN, C), lambda i: (0, i, 0)),
            blk,
            pl.BlockSpec((HID, HID), full), pl.BlockSpec((1, HID), full),
            pl.BlockSpec((1, 3 * HID), full),
            pl.BlockSpec((HID, HID), full), pl.BlockSpec((1, HID), full),
            pl.BlockSpec((HID, HID), full), pl.BlockSpec((1, HID), full),
            pl.BlockSpec((1, HID), full), pl.BlockSpec((1, HID), full),
        ],
        out_specs=blk,
        out_shape=jax.ShapeDtypeStruct((N, HID), jnp.float32),
    )(mp, wp, h, wsk, bsk.reshape(1, HID), p['beta'],
      w1, b1.reshape(1, HID), w2, b2.reshape(1, HID),
      g.reshape(1, HID), bln.reshape(1, HID))


def _pool_body(b_ref, h_ref, wp1_ref, bp1_ref, wp2_ref, bp2_ref, o_ref,
               acc, cacc, *, grid):
    i = pl.program_id(0)

    @pl.when(i == 0)
    def _():
        acc[...] = jnp.zeros_like(acc)
        cacc[...] = jnp.zeros_like(cacc)

    bids = b_ref[0, 0, :]
    gid = lax.broadcasted_iota(jnp.int32, (NG, bids.shape[0]), 0)
    onehot = (gid == bids[None, :]).astype(jnp.float32)
    acc[...] += lax.dot_general(onehot, h_ref[...], (((1,), (0,)), ((), ())),
                                preferred_element_type=jnp.float32)
    cnt = jnp.sum(onehot, axis=1, keepdims=True)
    cacc[...] += jnp.broadcast_to(cnt, (NG, HID))

    @pl.when(i == grid - 1)
    def _():
        gmean = acc[...] / jnp.maximum(cacc[...], 1.0)
        t = jnp.maximum(_mm(gmean, wp1_ref[...]) + bp1_ref[...], 0.0)
        o_ref[...] = _mm(t, wp2_ref[...]) + bp2_ref[0, 0]


def _pool_head(h, batch3, params):
    wp1, bp1 = params['pred1']
    wp2, bp2 = params['pred2']
    hid2 = wp1.shape[0]
    nt = wp2.shape[0]
    wp2p = jnp.pad(wp2, ((0, HID - nt), (0, 0)))
    bn = batch3.shape[2]
    grid = N // bn
    full = lambda i: (0, 0)
    res = pl.pallas_call(
        functools.partial(_pool_body, grid=grid),
        grid=(grid,),
        in_specs=[
            pl.BlockSpec((1, 1, bn), lambda i: (i, 0, 0)),
            pl.BlockSpec((bn, HID), lambda i: (i, 0)),
            pl.BlockSpec((hid2, HID), full),
            pl.BlockSpec((1, hid2), full),
            pl.BlockSpec((HID, hid2), full),
            pl.BlockSpec((1, nt), full),
        ],
        out_specs=pl.BlockSpec((NG, HID), lambda i: (0, 0)),
        out_shape=jax.ShapeDtypeStruct((NG, HID), jnp.float32),
        scratch_shapes=[pltpu.VMEM((NG, HID), jnp.float32),
                        pltpu.VMEM((NG, HID), jnp.float32)],
    )(batch3, h, wp1, bp1.reshape(1, hid2), wp2p, bp2.reshape(1, nt))
    return res[:, :nt]


def kernel(x, edge_index, edge_attr, batch, params):
    src = edge_index[0]
    dst = edge_index[1]
    h = _rowlin(x, params['node_enc'], relu=True, block=_BN)
    ea = _rowlin(edge_attr, params['edge_enc'], relu=False, block=_BE)
    zm = jnp.zeros((_NP, _HW), jnp.float32)
    zw = jnp.zeros((_NP, C), jnp.float32)
    batch3 = batch.reshape(10, 1, N // 10)
    for p in params['layers']:
        ea, e2 = _edge_layer(ea, p)
        q, k, v = _qkv(h, p)
        mp, wp = _sc_attn(q.reshape(_NC * N, _HW), k.reshape(_NC * N, _HW),
                          v.reshape(_NC * N, _HW), e2, src, dst, zm, zw)
        h = _post(mp, wp, h, p)
    return _pool_head(h, batch3, params)


# speedup vs baseline: 1.3820x; 1.0287x over previous
import functools

import jax
import jax.numpy as jnp
from jax import lax
from jax.experimental import pallas as pl
from jax.experimental.pallas import tpu as pltpu
from jax.experimental.pallas import tpu_sc as plsc

N = 10000
E = 160000
HID = 128
H = 8
C = 16
NG = 64

_NC = 2
_NS = 16
_NW = _NC * _NS
_CE = 128

_BN = 2000
_BE = 2000
_HW0 = HID // _NC


def _mm(a, w):
    return lax.dot_general(a, w, (((1,), (1,)), ((), ())),
                           preferred_element_type=jnp.float32)


def _rowlin_body(x_ref, w_ref, b_ref, o_ref, *, relu):
    y = _mm(x_ref[...], w_ref[...]) + b_ref[...]
    if relu:
        y = jnp.maximum(y, 0.0)
    o_ref[...] = y


def _rowlin(x, wb, relu, block):
    w, b = wb
    rows, din = x.shape
    dout = w.shape[0]
    grid = rows // block
    return pl.pallas_call(
        functools.partial(_rowlin_body, relu=relu),
        grid=(grid,),
        in_specs=[
            pl.BlockSpec((block, din), lambda i: (i, 0)),
            pl.BlockSpec((dout, din), lambda i: (0, 0)),
            pl.BlockSpec((1, dout), lambda i: (0, 0)),
        ],
        out_specs=pl.BlockSpec((block, dout), lambda i: (i, 0)),
        out_shape=jax.ShapeDtypeStruct((rows, dout), jnp.float32),
    )(x, w, b.reshape(1, dout))


def _edge_body(ea_ref, w1_ref, b1_ref, w2_ref, b2_ref, we_ref, ean_ref, e_ref):
    t = jnp.maximum(_mm(ea_ref[...], w1_ref[...]) + b1_ref[...], 0.0)
    ean = _mm(t, w2_ref[...]) + b2_ref[...]
    ean_ref[...] = ean
    e = _mm(ean, we_ref[...])
    e_ref[0] = e[:, :_HW0]
    e_ref[1] = e[:, _HW0:]


def _edge_layer(ea, p):
    w1, b1 = p['em1']
    w2, b2 = p['em2']
    we = p['e']
    grid = E // _BE
    full = lambda i: (0, 0)
    return pl.pallas_call(
        _edge_body,
        grid=(grid,),
        in_specs=[
            pl.BlockSpec((_BE, HID), lambda i: (i, 0)),
            pl.BlockSpec((HID, HID), full),
            pl.BlockSpec((1, HID), full),
            pl.BlockSpec((HID, HID), full),
            pl.BlockSpec((1, HID), full),
            pl.BlockSpec((HID, HID), full),
        ],
        out_specs=[
            pl.BlockSpec((_BE, HID), lambda i: (i, 0)),
            pl.BlockSpec((_NC, _BE, _HW0), lambda i: (0, i, 0)),
        ],
        out_shape=[
            jax.ShapeDtypeStruct((E, HID), jnp.float32),
            jax.ShapeDtypeStruct((_NC, E, _HW0), jnp.float32),
        ],
    )(ea, w1, b1.reshape(1, HID), w2, b2.reshape(1, HID), we)


def _qkv_body(h_ref, wq_ref, bq_ref, wk_ref, bk_ref, wv_ref, bv_ref,
              q_ref, kv_ref):
    h = h_ref[...]
    yq = _mm(h, wq_ref[...]) + bq_ref[...]
    q_ref[0] = yq[:, :_HW0]
    q_ref[1] = yq[:, _HW0:]
    yk = _mm(h, wk_ref[...]) + bk_ref[...]
    yv = _mm(h, wv_ref[...]) + bv_ref[...]
    for c in range(_NC):
        sl = slice(c * _HW0, (c + 1) * _HW0)
        kv_ref[c, :, :_HW0] = yk[:, sl]
        kv_ref[c, :, _HW0:] = yv[:, sl]


def _qkv(h, p):
    wq, bq = p['q']
    wk, bk = p['k']
    wv, bv = p['v']
    grid = N // _BN
    full = lambda i: (0, 0)
    blk = pl.BlockSpec((_BN, HID), lambda i: (i, 0))
    oblk = pl.BlockSpec((_NC, _BN, _HW0), lambda i: (0, i, 0))
    kvblk = pl.BlockSpec((_NC, _BN, HID), lambda i: (0, i, 0))
    return pl.pallas_call(
        _qkv_body,
        grid=(grid,),
        in_specs=[blk,
                  pl.BlockSpec((HID, HID), full), pl.BlockSpec((1, HID), full),
                  pl.BlockSpec((HID, HID), full), pl.BlockSpec((1, HID), full),
                  pl.BlockSpec((HID, HID), full), pl.BlockSpec((1, HID), full)],
        out_specs=[oblk, kvblk],
        out_shape=[jax.ShapeDtypeStruct((_NC, N, _HW0), jnp.float32),
                   jax.ShapeDtypeStruct((_NC, N, HID), jnp.float32)],
    )(h, wq, bq.reshape(1, HID), wk, bk.reshape(1, HID), wv, bv.reshape(1, HID))


_NCHUNKS = E // _CE

_GDN = lax.GatherDimensionNumbers(offset_dims=(), collapsed_slice_dims=(0,),
                                  start_index_map=(0,))


def _lane_shuffle(x, perm):
    return lax.gather(x, perm[:, None], _GDN, (1,),
                      mode=lax.GatherScatterMode.PROMISE_IN_BOUNDS)


def _lane_sum(x, perms):
    for perm in perms:
        x = x + _lane_shuffle(x, perm)
    return x


_NP = 10240
_ROWS_PER_S = _NP // _NS


_HH = H // _NC
_HW = _HH * C


def _sc_attn_body(q_hbm, kv_hbm, e_hbm, src_hbm, dst_hbm,
                  zm_hbm, zw_hbm, outm_hbm, outw_hbm,
                  srcb, dstb, idxb, idx2b, qb, kvb, msgb, eb, wb,
                  accm, accw, sem):
    cid = lax.axis_index("c")
    sid = lax.axis_index("s")

    r0 = pl.multiple_of(sid * _ROWS_PER_S, 8)
    pltpu.sync_copy(zm_hbm.at[pl.ds(r0, _ROWS_PER_S)],
                    accm.at[pl.ds(r0, _ROWS_PER_S)])
    pltpu.sync_copy(zw_hbm.at[pl.ds(r0, _ROWS_PER_S)],
                    accw.at[pl.ds(r0, _ROWS_PER_S)])
    plsc.subcore_barrier()

    base = _NCHUNKS // _NS
    extra = _NCHUNKS - base * _NS
    nch = base + jnp.where(sid < extra, 1, 0)
    lane = lax.iota(jnp.int32, 16)
    perms = [lane ^ m for m in (8, 4, 2, 1)]

    def chunk_body(t, carry):
        off = pl.multiple_of((sid + t * _NS) * _CE, _CE)
        pltpu.sync_copy(src_hbm.at[pl.ds(off, _CE)], srcb)
        pltpu.sync_copy(dst_hbm.at[pl.ds(off, _CE)], dstb)

        @plsc.parallel_loop(0, _CE // 16, unroll=4)
        def _(j):
            sl = pl.ds(j * 16, 16)
            idxb[sl] = srcb[sl] + cid * N
            idx2b[sl] = dstb[sl] + cid * N

        cp_kv = pltpu.async_copy(kv_hbm.at[idxb], kvb, sem)
        cp_q = pltpu.async_copy(q_hbm.at[idx2b], qb, sem)
        pltpu.sync_copy(e_hbm.at[cid, pl.ds(off, _CE)], eb)
        cp_kv.wait()
        cp_q.wait()

        @plsc.parallel_loop(0, _CE, unroll=4)
        def _(i):
            wacc = jnp.zeros((C,), jnp.float32)
            for hh in range(_HH):
                sl = pl.ds(hh * C, C)
                qh = qb[i, sl]
                eh = eb[i, sl]
                kj = kvb[i, sl] + eh
                a = _lane_sum(qh * kj, perms)
                wv = jnp.exp(a * 0.25)
                msgb[i, sl] = (kvb[i, pl.ds(_HW0 + hh * C, C)] + eh) * wv
                wacc = jnp.where(lane == hh, wv, wacc)
            wb[i, :] = wacc
        pltpu.sync_copy(msgb, accm.at[dstb], add=True)
        pltpu.sync_copy(wb, accw.at[dstb], add=True)
        return carry

    lax.fori_loop(0, nch, chunk_body, 0)
    plsc.subcore_barrier()

    pltpu.sync_copy(accm.at[pl.ds(r0, _ROWS_PER_S)],
                    outm_hbm.at[cid, pl.ds(r0, _ROWS_PER_S)])
    pltpu.sync_copy(accw.at[pl.ds(r0, _ROWS_PER_S)],
                    outw_hbm.at[cid, pl.ds(r0, _ROWS_PER_S)])


def _sc_attn(q2, kv2, e2, src, dst, zm, zw):
    mesh = plsc.VectorSubcoreMesh(core_axis_name="c", subcore_axis_name="s")
    fn = functools.partial(
        pl.kernel, mesh=mesh,
        compiler_params=pltpu.CompilerParams(use_tc_tiling_on_sc=False),
        out_type=[jax.ShapeDtypeStruct((_NC, _NP, _HW), jnp.float32),
                  jax.ShapeDtypeStruct((_NC, _NP, C), jnp.float32)],
        scratch_types=[
            pltpu.VMEM((_CE,), jnp.int32),
            pltpu.VMEM((_CE,), jnp.int32),
            pltpu.VMEM((_CE,), jnp.int32),
            pltpu.VMEM((_CE,), jnp.int32),
            pltpu.VMEM((_CE, _HW), jnp.float32),
            pltpu.VMEM((_CE, HID), jnp.float32),
            pltpu.VMEM((_CE, _HW), jnp.float32),
            pltpu.VMEM((_CE, _HW), jnp.float32),
            pltpu.VMEM((_CE, C), jnp.float32),
            pltpu.VMEM_SHARED((_NP, _HW), jnp.float32),
            pltpu.VMEM_SHARED((_NP, C), jnp.float32),
            pltpu.SemaphoreType.DMA,
        ],
    )(_sc_attn_body)
    return fn(q2, kv2, e2, src, dst, zm, zw)


def _post_body(mp_ref, wp_ref, h_ref, wsk_ref, bsk_ref, beta_ref,
               w1_ref, b1_ref, w2_ref, b2_ref, g_ref, bln_ref, o_ref):
    msg = jnp.concatenate([mp_ref[0], mp_ref[1]], axis=-1)
    wcat = jnp.concatenate([wp_ref[0], wp_ref[1]], axis=-1)
    rows = lax.broadcasted_iota(jnp.int32, (2 * C, HID), 0)
    gh = lax.broadcasted_iota(jnp.int32, (2 * C, HID), 1) // C
    expand = (((rows == gh) & (gh < _HH)) |
              ((rows == gh + 12) & (gh >= _HH))).astype(jnp.float32)
    den = lax.dot_general(wcat, expand, (((1,), (0,)), ((), ())),
                          preferred_element_type=jnp.float32)
    out = msg / (den + 1e-16)

    h = h_ref[...]
    xr = _mm(h, wsk_ref[...]) + bsk_ref[...]
    bvec = beta_ref[...]
    ba = bvec[:, 0:HID]
    bb = bvec[:, HID:2 * HID]
    bc = bvec[:, 2 * HID:3 * HID]
    logits = (_mm(out, ba) + _mm(xr, bb) + _mm(out - xr, bc))
    bt = jax.nn.sigmoid(logits)
    out = bt * xr + (1.0 - bt) * out

    t = jnp.maximum(_mm(out, w1_ref[...]) + b1_ref[...], 0.0)
    h2 = _mm(t, w2_ref[...]) + b2_ref[...] + h
    m = jnp.mean(h2, axis=-1, keepdims=True)
    d = h2 - m
    var = jnp.mean(d * d, axis=-1, keepdims=True)
    o_ref[...] = d * lax.rsqrt(var + 1e-5) * g_ref[...] + bln_ref[...]


def _post(mp, wp, h, p):
    wsk, bsk = p['skip']
    w1, b1 = p['nm1']
    w2, b2 = p['nm2']
    g, bln = p['ln']
    grid = N // _BN
    full = lambda i: (0, 0)
    blk = pl.BlockSpec((_BN, HID), lambda i: (i, 0))
    return pl.pallas_call(
        _post_body,
        grid=(grid,),
        in_specs=[
            pl.BlockSpec((_NC, _BN, _HW), lambda i: (0, i, 0)),
            pl.BlockSpec((_NC, _BN, C), lambda i: (0, i, 0)),
            blk,
            pl.BlockSpec((HID, HID), full), pl.BlockSpec((1, HID), full),
            pl.BlockSpec((1, 3 * HID), full),
            pl.BlockSpec((HID, HID), full), pl.BlockSpec((1, HID), full),
            pl.BlockSpec((HID, HID), full), pl.BlockSpec((1, HID), full),
            pl.BlockSpec((1, HID), full), pl.BlockSpec((1, HID), full),
        ],
        out_specs=blk,
        out_shape=jax.ShapeDtypeStruct((N, HID), jnp.float32),
    )(mp, wp, h, wsk, bsk.reshape(1, HID), p['beta'],
      w1, b1.reshape(1, HID), w2, b2.reshape(1, HID),
      g.reshape(1, HID), bln.reshape(1, HID))


def _pool_body(b_ref, h_ref, wp1_ref, bp1_ref, wp2_ref, bp2_ref, o_ref,
               acc, cacc, *, grid):
    i = pl.program_id(0)

    @pl.when(i == 0)
    def _():
        acc[...] = jnp.zeros_like(acc)
        cacc[...] = jnp.zeros_like(cacc)

    bids = b_ref[0, 0, :]
    gid = lax.broadcasted_iota(jnp.int32, (NG, bids.shape[0]), 0)
    onehot = (gid == bids[None, :]).astype(jnp.float32)
    acc[...] += lax.dot_general(onehot, h_ref[...], (((1,), (0,)), ((), ())),
                                preferred_element_type=jnp.float32)
    cnt = jnp.sum(onehot, axis=1, keepdims=True)
    cacc[...] += jnp.broadcast_to(cnt, (NG, HID))

    @pl.when(i == grid - 1)
    def _():
        gmean = acc[...] / jnp.maximum(cacc[...], 1.0)
        t = jnp.maximum(_mm(gmean, wp1_ref[...]) + bp1_ref[...], 0.0)
        o_ref[...] = _mm(t, wp2_ref[...]) + bp2_ref[0, 0]


def _pool_head(h, batch3, params):
    wp1, bp1 = params['pred1']
    wp2, bp2 = params['pred2']
    hid2 = wp1.shape[0]
    nt = wp2.shape[0]
    wp2p = jnp.pad(wp2, ((0, HID - nt), (0, 0)))
    bn = batch3.shape[2]
    grid = N // bn
    full = lambda i: (0, 0)
    res = pl.pallas_call(
        functools.partial(_pool_body, grid=grid),
        grid=(grid,),
        in_specs=[
            pl.BlockSpec((1, 1, bn), lambda i: (i, 0, 0)),
            pl.BlockSpec((bn, HID), lambda i: (i, 0)),
            pl.BlockSpec((hid2, HID), full),
            pl.BlockSpec((1, hid2), full),
            pl.BlockSpec((HID, hid2), full),
            pl.BlockSpec((1, nt), full),
        ],
        out_specs=pl.BlockSpec((NG, HID), lambda i: (0, 0)),
        out_shape=jax.ShapeDtypeStruct((NG, HID), jnp.float32),
        scratch_shapes=[pltpu.VMEM((NG, HID), jnp.float32),
                        pltpu.VMEM((NG, HID), jnp.float32)],
    )(batch3, h, wp1, bp1.reshape(1, hid2), wp2p, bp2.reshape(1, nt))
    return res[:, :nt]


def kernel(x, edge_index, edge_attr, batch, params):
    src = edge_index[0]
    dst = edge_index[1]
    h = _rowlin(x, params['node_enc'], relu=True, block=_BN)
    ea = _rowlin(edge_attr, params['edge_enc'], relu=False, block=_BE)
    zm = jnp.zeros((_NP, _HW), jnp.float32)
    zw = jnp.zeros((_NP, C), jnp.float32)
    batch3 = batch.reshape(10, 1, N // 10)
    for p in params['layers']:
        ea, e2 = _edge_layer(ea, p)
        q, kv = _qkv(h, p)
        mp, wp = _sc_attn(q.reshape(_NC * N, _HW), kv.reshape(_NC * N, HID),
                          e2, src, dst, zm, zw)
        h = _post(mp, wp, h, p)
    return _pool_head(h, batch3, params)
